# Initial kernel scaffold; baseline (speedup 1.0000x reference)
import jax
import jax.numpy as jnp
from jax.experimental import pallas as pl


def kernel(x, edge_index, W_in, b_in, conv_weights, W_out, b_out):
    raise NotImplementedError("write your pallas kernel here")


# speedup vs baseline: 6.0032x; 6.0032x over previous
import functools
import math

import jax
import jax.numpy as jnp
from jax import lax
from jax.experimental import pallas as pl
from jax.experimental.pallas import tpu as pltpu
from jax.experimental.pallas import tpu_sc as plsc

N_NODES = 10000
D = 128
NUM_LAYERS = 8
ALPHA = 0.1
THETA = 0.5

NC = 2
NS = 16
NW = NC * NS
CHUNK = 128
N_PAD = 10240
ROWS_PER_SUB = N_PAD // NS
BLK = 1280
GRID = N_PAD // BLK

_mesh = plsc.VectorSubcoreMesh(core_axis_name="c", subcore_axis_name="s")


STRIPE = 8


def _n_chunks(e_total):
    nchunk = -(-e_total // (NW * CHUNK))
    return -(-nchunk // STRIPE) * STRIPE


def _make_deg_kernel(nchunk):
    @functools.partial(
        pl.kernel,
        out_type=jax.ShapeDtypeStruct((NC, N_PAD, 16), jnp.float32),
        mesh=_mesh,
        scratch_types=[
            pltpu.VMEM_SHARED((N_PAD, 16), jnp.float32),
            pltpu.VMEM((nchunk, CHUNK), jnp.int32),
            pltpu.VMEM((CHUNK, 16), jnp.float32),
        ],
    )
    def deg_kernel(col_hbm, zeros_hbm, out_hbm, acc, colv, onesv):
        c = lax.axis_index("c")
        s = lax.axis_index("s")
        wid = c * NS + s
        pltpu.sync_copy(zeros_hbm, acc.at[pl.ds(s * ROWS_PER_SUB, ROWS_PER_SUB)])
        pltpu.sync_copy(col_hbm.at[wid], colv)

        def fill(i, carry):
            onesv[i] = jnp.full((16,), 1.0, jnp.float32)
            return carry

        lax.fori_loop(0, CHUNK, fill, 0)
        plsc.subcore_barrier()

        def body(j, carry):
            pltpu.sync_copy(onesv, acc.at[colv.at[j]], add=True)
            return carry

        lax.fori_loop(0, nchunk, body, 0)
        plsc.subcore_barrier()
        sl = pl.ds(s * ROWS_PER_SUB, ROWS_PER_SUB)
        pltpu.sync_copy(acc.at[sl], out_hbm.at[c].at[sl])

    return deg_kernel


def _make_prop_kernel(nchunk):
    @functools.partial(
        pl.kernel,
        out_type=jax.ShapeDtypeStruct((NC, N_PAD, D), jnp.float32),
        mesh=_mesh,
        scratch_types=[
            pltpu.VMEM_SHARED((N_PAD, D), jnp.float32),
            pltpu.VMEM((STRIPE, CHUNK), jnp.int32),
            pltpu.VMEM((STRIPE, CHUNK), jnp.int32),
            pltpu.VMEM((2, CHUNK, D), jnp.float32),
            pltpu.SemaphoreType.DMA((2,)),
        ],
    )
    def prop_kernel(g_hbm, row_hbm, col_hbm, zeros_hbm, out_hbm,
                    acc, rowv, colv, msg, sems):
        c = lax.axis_index("c")
        s = lax.axis_index("s")
        wid = c * NS + s
        pltpu.sync_copy(zeros_hbm, acc.at[pl.ds(s * ROWS_PER_SUB, ROWS_PER_SUB)])
        plsc.subcore_barrier()

        def stripe_body(t, carry):
            base = t * STRIPE
            pltpu.sync_copy(row_hbm.at[wid].at[pl.ds(base, STRIPE)], rowv)
            pltpu.sync_copy(col_hbm.at[wid].at[pl.ds(base, STRIPE)], colv)
            pltpu.async_copy(g_hbm.at[rowv.at[0]], msg.at[0], sems.at[0])

            def body(j, carry2):
                cb = lax.rem(j, 2)
                nb = lax.rem(j + 1, 2)

                @pl.when(j + 1 < STRIPE)
                def _():
                    pltpu.async_copy(g_hbm.at[rowv.at[j + 1]], msg.at[nb],
                                     sems.at[nb])

                pltpu.make_async_copy(g_hbm.at[rowv.at[j]], msg.at[cb],
                                      sems.at[cb]).wait()
                pltpu.sync_copy(msg.at[cb], acc.at[colv.at[j]], add=True)
                return carry2

            lax.fori_loop(0, STRIPE, body, 0)
            return carry

        lax.fori_loop(0, nchunk // STRIPE, stripe_body, 0)
        plsc.subcore_barrier()
        sl = pl.ds(s * ROWS_PER_SUB, ROWS_PER_SUB)
        pltpu.sync_copy(acc.at[sl], out_hbm.at[c].at[sl])

    return prop_kernel


def _row_spec():
    return pl.BlockSpec((BLK, D), lambda i: (i, 0))


def _full_spec(shape):
    return pl.BlockSpec(shape, lambda i: tuple(0 for _ in shape))


def _prologue_body(x_ref, w_ref, b_ref, d0_ref, d1_ref,
                   x0_ref, g_ref, dis_ref):
    deg = d0_ref[:, 0:1] + d1_ref[:, 0:1] + 1.0
    dis = lax.rsqrt(deg)
    x0 = jnp.maximum(
        jnp.dot(x_ref[...], w_ref[...], preferred_element_type=jnp.float32)
        + b_ref[...], 0.0)
    x0_ref[...] = x0
    g_ref[...] = x0 * dis
    dis_ref[...] = dis


def _prologue(x_pad, w_in, b_in, d0, d1):
    return pl.pallas_call(
        _prologue_body,
        grid=(GRID,),
        in_specs=[_row_spec(), _full_spec((D, D)), _full_spec((1, D)),
                  pl.BlockSpec((BLK, 16), lambda i: (i, 0)),
                  pl.BlockSpec((BLK, 16), lambda i: (i, 0))],
        out_specs=[_row_spec(), _row_spec(),
                   pl.BlockSpec((BLK, 1), lambda i: (i, 0))],
        out_shape=[jax.ShapeDtypeStruct((N_PAD, D), jnp.float32),
                   jax.ShapeDtypeStruct((N_PAD, D), jnp.float32),
                   jax.ShapeDtypeStruct((N_PAD, 1), jnp.float32)],
    )(x_pad, w_in, b_in, d0, d1)


def _layer_body(p0_ref, p1_ref, g_ref, x0_ref, dis_ref, wf_ref, out_ref):
    dis = dis_ref[...]
    sv = (1.0 - ALPHA) * dis * (p0_ref[...] + p1_ref[...] + g_ref[...]) \
        + ALPHA * x0_ref[...]
    h = jnp.maximum(
        jnp.dot(sv, wf_ref[...], preferred_element_type=jnp.float32), 0.0)
    out_ref[...] = h * dis


def _layer(p0, p1, g, x0, dis, wf):
    return pl.pallas_call(
        _layer_body,
        grid=(GRID,),
        in_specs=[_row_spec(), _row_spec(), _row_spec(), _row_spec(),
                  pl.BlockSpec((BLK, 1), lambda i: (i, 0)),
                  _full_spec((D, D))],
        out_specs=_row_spec(),
        out_shape=jax.ShapeDtypeStruct((N_PAD, D), jnp.float32),
    )(p0, p1, g, x0, dis, wf)


def _final_body(p0_ref, p1_ref, g_ref, x0_ref, dis_ref, wf_ref,
                wo_ref, bo_ref, out_ref):
    dis = dis_ref[...]
    sv = (1.0 - ALPHA) * dis * (p0_ref[...] + p1_ref[...] + g_ref[...]) \
        + ALPHA * x0_ref[...]
    h = jnp.maximum(
        jnp.dot(sv, wf_ref[...], preferred_element_type=jnp.float32), 0.0)
    out_ref[...] = jnp.dot(h, wo_ref[...],
                           preferred_element_type=jnp.float32) + bo_ref[...]


def _final(p0, p1, g, x0, dis, wf, w_out, b_out):
    return pl.pallas_call(
        _final_body,
        grid=(GRID,),
        in_specs=[_row_spec(), _row_spec(), _row_spec(), _row_spec(),
                  pl.BlockSpec((BLK, 1), lambda i: (i, 0)),
                  _full_spec((D, D)), _full_spec((D, 1)), _full_spec((1, 1))],
        out_specs=pl.BlockSpec((BLK, 1), lambda i: (i, 0)),
        out_shape=jax.ShapeDtypeStruct((N_PAD, 1), jnp.float32),
    )(p0, p1, g, x0, dis, wf, w_out, b_out)


def kernel(x, edge_index, W_in, b_in, conv_weights, W_out, b_out):
    n, d = x.shape
    e = edge_index.shape[1]
    nchunk = _n_chunks(e)
    e_pad = NW * nchunk * CHUNK

    row = edge_index[0].astype(jnp.int32)
    col = edge_index[1].astype(jnp.int32)
    row3 = jnp.pad(row, (0, e_pad - e), constant_values=N_NODES) \
        .reshape(NW, nchunk, CHUNK)
    col3 = jnp.pad(col, (0, e_pad - e), constant_values=N_NODES) \
        .reshape(NW, nchunk, CHUNK)
    x_pad = jnp.pad(x, ((0, N_PAD - n), (0, 0)))
    zeros16 = jnp.zeros((ROWS_PER_SUB, 16), jnp.float32)
    zeros128 = jnp.zeros((ROWS_PER_SUB, D), jnp.float32)

    eye = jnp.eye(D, dtype=jnp.float32)
    betas = [math.log(THETA / (i + 1) + 1.0) for i in range(NUM_LAYERS)]
    wfs = [(1.0 - b) * eye + b * conv_weights[i] for i, b in enumerate(betas)]

    deg_kernel = _make_deg_kernel(nchunk)
    prop_kernel = _make_prop_kernel(nchunk)

    degp = deg_kernel(col3, zeros16)
    x0, g, dis = _prologue(x_pad, W_in, b_in.reshape(1, D), degp[0], degp[1])

    for i in range(NUM_LAYERS - 1):
        p = prop_kernel(g, row3, col3, zeros128)
        g = _layer(p[0], p[1], g, x0, dis, wfs[i])
    p = prop_kernel(g, row3, col3, zeros128)
    out = _final(p[0], p[1], g, x0, dis, wfs[NUM_LAYERS - 1],
                 W_out, b_out.reshape(1, 1))
    return out[:N_NODES, 0]


# speedup vs baseline: 6.0059x; 1.0004x over previous
import functools
import math

import jax
import jax.numpy as jnp
from jax import lax
from jax.experimental import pallas as pl
from jax.experimental.pallas import tpu as pltpu
from jax.experimental.pallas import tpu_sc as plsc

N_NODES = 10000
D = 128
NUM_LAYERS = 8
ALPHA = 0.1
THETA = 0.5

NC = 2
NS = 16
NW = NC * NS
CHUNK = 128
N_PAD = 10240
ROWS_PER_SUB = N_PAD // NS
BLK = 1280
GRID = N_PAD // BLK

_mesh = plsc.VectorSubcoreMesh(core_axis_name="c", subcore_axis_name="s")


STRIPE = 8


def _n_chunks(e_total):
    nchunk = -(-e_total // (NW * CHUNK))
    return -(-nchunk // STRIPE) * STRIPE


def _make_deg_kernel(nchunk):
    @functools.partial(
        pl.kernel,
        out_type=jax.ShapeDtypeStruct((NC, N_PAD, 16), jnp.float32),
        mesh=_mesh,
        scratch_types=[
            pltpu.VMEM_SHARED((N_PAD, 16), jnp.float32),
            pltpu.VMEM((nchunk, CHUNK), jnp.int32),
            pltpu.VMEM((CHUNK, 16), jnp.float32),
        ],
    )
    def deg_kernel(col_hbm, zeros_hbm, out_hbm, acc, colv, onesv):
        c = lax.axis_index("c")
        s = lax.axis_index("s")
        wid = c * NS + s
        pltpu.sync_copy(zeros_hbm, acc.at[pl.ds(s * ROWS_PER_SUB, ROWS_PER_SUB)])
        pltpu.sync_copy(col_hbm.at[wid], colv)

        def fill(i, carry):
            onesv[i] = jnp.full((16,), 1.0, jnp.float32)
            return carry

        lax.fori_loop(0, CHUNK, fill, 0)
        plsc.subcore_barrier()

        def body(j, carry):
            pltpu.sync_copy(onesv, acc.at[colv.at[j]], add=True)
            return carry

        lax.fori_loop(0, nchunk, body, 0)
        plsc.subcore_barrier()
        sl = pl.ds(s * ROWS_PER_SUB, ROWS_PER_SUB)
        pltpu.sync_copy(acc.at[sl], out_hbm.at[c].at[sl])

    return deg_kernel


def _make_prop_kernel(nchunk):
    @functools.partial(
        pl.kernel,
        out_type=jax.ShapeDtypeStruct((NC, N_PAD, D), jnp.float32),
        mesh=_mesh,
        scratch_types=[
            pltpu.VMEM_SHARED((N_PAD, D), jnp.float32),
            pltpu.VMEM((3 * STRIPE, CHUNK), jnp.int32),
            pltpu.VMEM((3 * STRIPE, CHUNK), jnp.int32),
            pltpu.VMEM((2, CHUNK, D), jnp.float32),
            pltpu.SemaphoreType.DMA((2,)),
            pltpu.SemaphoreType.DMA((2,)),
            pltpu.SemaphoreType.DMA((6,)),
        ],
    )
    def prop_kernel(g_hbm, row_hbm, col_hbm, zeros_hbm, out_hbm,
                    acc, rowv, colv, msg, gsems, ssems, isems):
        c = lax.axis_index("c")
        s = lax.axis_index("s")
        wid = c * NS + s
        nstripe = nchunk // STRIPE

        def issue_idx(t, tb):
            sl = pl.ds(t * STRIPE, STRIPE)
            dl = pl.ds(tb * STRIPE, STRIPE)
            pltpu.async_copy(row_hbm.at[wid].at[sl], rowv.at[dl],
                             isems.at[2 * tb])
            pltpu.async_copy(col_hbm.at[wid].at[sl], colv.at[dl],
                             isems.at[2 * tb + 1])

        def wait_idx(t, tb):
            sl = pl.ds(t * STRIPE, STRIPE)
            dl = pl.ds(tb * STRIPE, STRIPE)
            pltpu.make_async_copy(row_hbm.at[wid].at[sl], rowv.at[dl],
                                  isems.at[2 * tb]).wait()
            pltpu.make_async_copy(col_hbm.at[wid].at[sl], colv.at[dl],
                                  isems.at[2 * tb + 1]).wait()

        issue_idx(0, 0)
        pltpu.sync_copy(zeros_hbm, acc.at[pl.ds(s * ROWS_PER_SUB, ROWS_PER_SUB)])
        wait_idx(0, 0)

        @pl.when(nstripe > 1)
        def _():
            issue_idx(1, 1)

        plsc.subcore_barrier()
        pltpu.async_copy(g_hbm.at[rowv.at[0]], msg.at[0], gsems.at[0])

        def body(j, carry):
            b = lax.rem(j, 2)
            nb = lax.rem(j + 1, 2)
            r = lax.rem(lax.div(j, STRIPE), 3) * STRIPE + lax.rem(j, STRIPE)

            pltpu.make_async_copy(g_hbm.at[rowv.at[r]], msg.at[b],
                                  gsems.at[b]).wait()

            j1 = j + 1
            jj1 = lax.rem(j1, STRIPE)
            t1 = lax.div(j1, STRIPE)
            tb1 = lax.rem(t1, 3)
            r1 = tb1 * STRIPE + jj1

            @pl.when((jj1 == 0) & (j1 < nchunk))
            def _():
                wait_idx(t1, tb1)

                @pl.when(t1 + 1 < nstripe)
                def _():
                    issue_idx(t1 + 1, lax.rem(t1 + 1, 3))

            @pl.when(j1 < nchunk)
            def _():
                pltpu.async_copy(g_hbm.at[rowv.at[r1]], msg.at[nb],
                                 gsems.at[nb])

            pltpu.sync_copy(msg.at[b], acc.at[colv.at[r]], add=True)
            return carry

        lax.fori_loop(0, nchunk, body, 0)
        plsc.subcore_barrier()
        sl = pl.ds(s * ROWS_PER_SUB, ROWS_PER_SUB)
        pltpu.sync_copy(acc.at[sl], out_hbm.at[c].at[sl])

    return prop_kernel


def _row_spec():
    return pl.BlockSpec((BLK, D), lambda i: (i, 0))


def _full_spec(shape):
    return pl.BlockSpec(shape, lambda i: tuple(0 for _ in shape))


def _prologue_body(x_ref, w_ref, b_ref, d0_ref, d1_ref,
                   x0_ref, g_ref, dis_ref):
    deg = d0_ref[:, 0:1] + d1_ref[:, 0:1] + 1.0
    dis = lax.rsqrt(deg)
    x0 = jnp.maximum(
        jnp.dot(x_ref[...], w_ref[...], preferred_element_type=jnp.float32)
        + b_ref[...], 0.0)
    x0_ref[...] = x0
    g_ref[...] = x0 * dis
    dis_ref[...] = dis


def _prologue(x_pad, w_in, b_in, d0, d1):
    return pl.pallas_call(
        _prologue_body,
        grid=(GRID,),
        in_specs=[_row_spec(), _full_spec((D, D)), _full_spec((1, D)),
                  pl.BlockSpec((BLK, 16), lambda i: (i, 0)),
                  pl.BlockSpec((BLK, 16), lambda i: (i, 0))],
        out_specs=[_row_spec(), _row_spec(),
                   pl.BlockSpec((BLK, 1), lambda i: (i, 0))],
        out_shape=[jax.ShapeDtypeStruct((N_PAD, D), jnp.float32),
                   jax.ShapeDtypeStruct((N_PAD, D), jnp.float32),
                   jax.ShapeDtypeStruct((N_PAD, 1), jnp.float32)],
    )(x_pad, w_in, b_in, d0, d1)


def _layer_body(p0_ref, p1_ref, g_ref, x0_ref, dis_ref, wf_ref, out_ref):
    dis = dis_ref[...]
    sv = (1.0 - ALPHA) * dis * (p0_ref[...] + p1_ref[...] + g_ref[...]) \
        + ALPHA * x0_ref[...]
    h = jnp.maximum(
        jnp.dot(sv, wf_ref[...], preferred_element_type=jnp.float32), 0.0)
    out_ref[...] = h * dis


def _layer(p0, p1, g, x0, dis, wf):
    return pl.pallas_call(
        _layer_body,
        grid=(GRID,),
        in_specs=[_row_spec(), _row_spec(), _row_spec(), _row_spec(),
                  pl.BlockSpec((BLK, 1), lambda i: (i, 0)),
                  _full_spec((D, D))],
        out_specs=_row_spec(),
        out_shape=jax.ShapeDtypeStruct((N_PAD, D), jnp.float32),
    )(p0, p1, g, x0, dis, wf)


def _final_body(p0_ref, p1_ref, g_ref, x0_ref, dis_ref, wf_ref,
                wo_ref, bo_ref, out_ref):
    dis = dis_ref[...]
    sv = (1.0 - ALPHA) * dis * (p0_ref[...] + p1_ref[...] + g_ref[...]) \
        + ALPHA * x0_ref[...]
    h = jnp.maximum(
        jnp.dot(sv, wf_ref[...], preferred_element_type=jnp.float32), 0.0)
    out_ref[...] = jnp.dot(h, wo_ref[...],
                           preferred_element_type=jnp.float32) + bo_ref[...]


def _final(p0, p1, g, x0, dis, wf, w_out, b_out):
    return pl.pallas_call(
        _final_body,
        grid=(GRID,),
        in_specs=[_row_spec(), _row_spec(), _row_spec(), _row_spec(),
                  pl.BlockSpec((BLK, 1), lambda i: (i, 0)),
                  _full_spec((D, D)), _full_spec((D, 1)), _full_spec((1, 1))],
        out_specs=pl.BlockSpec((BLK, 1), lambda i: (i, 0)),
        out_shape=jax.ShapeDtypeStruct((N_PAD, 1), jnp.float32),
    )(p0, p1, g, x0, dis, wf, w_out, b_out)


def kernel(x, edge_index, W_in, b_in, conv_weights, W_out, b_out):
    n, d = x.shape
    e = edge_index.shape[1]
    nchunk = _n_chunks(e)
    e_pad = NW * nchunk * CHUNK

    row = edge_index[0].astype(jnp.int32)
    col = edge_index[1].astype(jnp.int32)
    row3 = jnp.pad(row, (0, e_pad - e), constant_values=N_NODES) \
        .reshape(NW, nchunk, CHUNK)
    col3 = jnp.pad(col, (0, e_pad - e), constant_values=N_NODES) \
        .reshape(NW, nchunk, CHUNK)
    x_pad = jnp.pad(x, ((0, N_PAD - n), (0, 0)))
    zeros16 = jnp.zeros((ROWS_PER_SUB, 16), jnp.float32)
    zeros128 = jnp.zeros((ROWS_PER_SUB, D), jnp.float32)

    eye = jnp.eye(D, dtype=jnp.float32)
    betas = [math.log(THETA / (i + 1) + 1.0) for i in range(NUM_LAYERS)]
    wfs = [(1.0 - b) * eye + b * conv_weights[i] for i, b in enumerate(betas)]

    deg_kernel = _make_deg_kernel(nchunk)
    prop_kernel = _make_prop_kernel(nchunk)

    degp = deg_kernel(col3, zeros16)
    x0, g, dis = _prologue(x_pad, W_in, b_in.reshape(1, D), degp[0], degp[1])

    for i in range(NUM_LAYERS - 1):
        p = prop_kernel(g, row3, col3, zeros128)
        g = _layer(p[0], p[1], g, x0, dis, wfs[i])
    p = prop_kernel(g, row3, col3, zeros128)
    out = _final(p[0], p[1], g, x0, dis, wfs[NUM_LAYERS - 1],
                 W_out, b_out.reshape(1, 1))
    return out[:N_NODES, 0]
